# baseline (speedup 1.0000x reference)
import functools

import jax
import jax.numpy as jnp
from jax import lax
from jax.experimental import pallas as pl
from jax.experimental.pallas import tpu as pltpu
from jax.experimental.pallas import tpu_sc as plsc

_D = 128
_B = 4096
_S = 50
_LE = 20
_NW = 32
_CHUNK = 128

_NTEXT = _B * _S
_TCHUNKS = _NTEXT // _CHUNK
_TCH_W = _TCHUNKS // _NW
_EB_W = _B // _NW
_NROWS = _NTEXT + _B


def _sc_gather(table, tidx3d, eidx3d):
    mesh = plsc.VectorSubcoreMesh(core_axis_name="c", subcore_axis_name="s")

    @functools.partial(
        pl.kernel,
        mesh=mesh,
        out_type=jax.ShapeDtypeStruct((_NROWS, _D), jnp.float32),
        scratch_types=[
            pltpu.VMEM((_TCH_W, _CHUNK), jnp.int32),
            pltpu.VMEM((_LE, _CHUNK), jnp.int32),
            pltpu.VMEM((_CHUNK, _D), jnp.float32),
            pltpu.VMEM((_EB_W, _D), jnp.float32),
            pltpu.SemaphoreType.DMA,
        ],
    )
    def gather_kernel(table_hbm, tidx_hbm, eidx_hbm, out_hbm,
                      idx_v, eidx_v, rows_v, acc_v, sem):
        wid = lax.axis_index("s") * 2 + lax.axis_index("c")

        pltpu.sync_copy(tidx_hbm.at[wid], idx_v)

        def tbody(j, carry):
            pltpu.async_copy(table_hbm.at[idx_v.at[j]], rows_v, sem).wait()
            pltpu.sync_copy(
                rows_v, out_hbm.at[pl.ds((wid * _TCH_W + j) * _CHUNK, _CHUNK)])
            return carry

        lax.fori_loop(0, _TCH_W, tbody, 0)

        pltpu.sync_copy(eidx_hbm.at[wid], eidx_v)
        pltpu.async_copy(table_hbm.at[eidx_v.at[0]], acc_v, sem).wait()

        def ebody(l, carry):
            pltpu.async_copy(table_hbm.at[eidx_v.at[l]], acc_v, sem,
                             add=True).wait()
            return carry

        lax.fori_loop(1, _LE, ebody, 0)
        pltpu.sync_copy(acc_v, out_hbm.at[pl.ds(_NTEXT + wid * _EB_W, _EB_W)])

    return gather_kernel(table, tidx3d, eidx3d)


def _leaky(x):
    return jnp.maximum(x, 0.01 * x)


def _mlp_body(x_ref, w1_ref, b1_ref, w2_ref, b2_ref, w3_ref, b3_ref, o_ref):
    x = x_ref[...]
    h = _leaky(jnp.dot(x, w1_ref[...], preferred_element_type=jnp.float32)
               + b1_ref[...])
    h = _leaky(jnp.dot(h, w2_ref[...], preferred_element_type=jnp.float32)
               + b2_ref[...])
    h = _leaky(jnp.dot(h, w3_ref[...], preferred_element_type=jnp.float32)
               + b3_ref[...])
    o_ref[:, 0, :] = h[:, 0:30]
    o_ref[:, 1, :] = h[:, 128:158]


def _tc_mlp(rows2, w1d, b1d, w2d, b2d, w3d, b3d):
    n2 = rows2.shape[0]
    blk = 2048
    grid = (n2 // blk,)
    wspec = pl.BlockSpec((256, 256), lambda i: (0, 0))
    bspec = pl.BlockSpec((1, 256), lambda i: (0, 0))
    return pl.pallas_call(
        _mlp_body,
        grid=grid,
        in_specs=[
            pl.BlockSpec((blk, 256), lambda i: (i, 0)),
            wspec, bspec, wspec, bspec, wspec, bspec,
        ],
        out_specs=pl.BlockSpec((blk, 2, 30), lambda i: (i, 0, 0)),
        out_shape=jax.ShapeDtypeStruct((n2, 2, 30), jnp.float32),
    )(rows2, w1d, b1d, w2d, b2d, w3d, b3d)


def _blockdiag(w, fan_in, fan_out):
    wp = jnp.zeros((_D, _D), jnp.float32).at[:fan_in, :fan_out].set(w)
    wd = jnp.zeros((256, 256), jnp.float32)
    return wd.at[:_D, :_D].set(wp).at[_D:, _D:].set(wp)


def kernel(textC, emoC, tableC, W1, b1, W2, b2, W3, b3):
    textC = textC.astype(jnp.int32)
    emoC = emoC.astype(jnp.int32)

    tidx3d = textC.reshape(_NW, _TCH_W, _CHUNK)
    eidx3d = emoC.T.reshape(_LE, _NW, _EB_W).transpose(1, 0, 2)

    rows = _sc_gather(tableC, tidx3d, eidx3d)

    w1d = _blockdiag(W1, _D, 100)
    w2d = _blockdiag(W2, 100, 60)
    w3d = _blockdiag(W3, 60, 30)

    def bpad(b, width):
        bp = jnp.zeros((1, _D), jnp.float32).at[0, :width].set(b)
        return jnp.concatenate([bp, bp], axis=1)

    b1d = bpad(b1, 100)
    b2d = bpad(b2, 60)
    b3d = bpad(b3, 30)

    out = _tc_mlp(rows.reshape(_NROWS // 2, 256), w1d, b1d, w2d, b2d, w3d, b3d)

    out = out.reshape(_NROWS, 30)
    outputsC = out[:_NTEXT].reshape(_B, _S, 30)
    emo_out = out[_NTEXT:].reshape(_B, 1, 30)
    return (outputsC, emo_out)


# speedup vs baseline: 1.1188x; 1.1188x over previous
import functools

import jax
import jax.numpy as jnp
from jax import lax
from jax.experimental import pallas as pl
from jax.experimental.pallas import tpu as pltpu
from jax.experimental.pallas import tpu_sc as plsc

_D = 128
_B = 4096
_S = 50
_LE = 20
_NW = 32
_CHUNK = 128

_NTEXT = _B * _S
_TCHUNKS = _NTEXT // _CHUNK
_TCH_W = _TCHUNKS // _NW
_EB_W = _B // _NW
_NROWS = _NTEXT + _B


def _sc_gather(table, tidx3d, eidx3d):
    mesh = plsc.VectorSubcoreMesh(core_axis_name="c", subcore_axis_name="s")

    @functools.partial(
        pl.kernel,
        mesh=mesh,
        out_type=jax.ShapeDtypeStruct((_NROWS, _D), jnp.float32),
        scratch_types=[
            pltpu.VMEM((_TCH_W, _CHUNK), jnp.int32),
            pltpu.VMEM((_LE, _CHUNK), jnp.int32),
            pltpu.VMEM((_CHUNK, _D), jnp.float32),
            pltpu.VMEM((_EB_W, _D), jnp.float32),
            pltpu.SemaphoreType.DMA,
        ],
    )
    def gather_kernel(table_hbm, tidx_hbm, eidx_hbm, out_hbm,
                      idx_v, eidx_v, rows_v, acc_v, sem):
        wid = lax.axis_index("s") * 2 + lax.axis_index("c")

        pltpu.sync_copy(tidx_hbm.at[wid], idx_v)

        def tbody(j, carry):
            pltpu.async_copy(table_hbm.at[idx_v.at[j]], rows_v, sem).wait()
            pltpu.sync_copy(
                rows_v, out_hbm.at[pl.ds((wid * _TCH_W + j) * _CHUNK, _CHUNK)])
            return carry

        lax.fori_loop(0, _TCH_W, tbody, 0)

        pltpu.sync_copy(eidx_hbm.at[wid], eidx_v)
        pltpu.async_copy(table_hbm.at[eidx_v.at[0]], acc_v, sem).wait()

        def ebody(l, carry):
            pltpu.async_copy(table_hbm.at[eidx_v.at[l]], acc_v, sem,
                             add=True).wait()
            return carry

        lax.fori_loop(1, _LE, ebody, 0)
        pltpu.sync_copy(acc_v, out_hbm.at[pl.ds(_NTEXT + wid * _EB_W, _EB_W)])

    return gather_kernel(table, tidx3d, eidx3d)


def _leaky(x):
    return jnp.maximum(x, 0.01 * x)


def _mlp_body(x_ref, w1_ref, b1_ref, w2_ref, b2_ref, w3_ref, b3_ref, o_ref):
    x = x_ref[...]
    h = _leaky(jnp.dot(x, w1_ref[...], preferred_element_type=jnp.float32)
               + b1_ref[...])
    h = _leaky(jnp.dot(h, w2_ref[...], preferred_element_type=jnp.float32)
               + b2_ref[...])
    h = _leaky(jnp.dot(h, w3_ref[...], preferred_element_type=jnp.float32)
               + b3_ref[...])
    o_ref[:, 0, :] = h[:, 0:30]
    o_ref[:, 1, :] = h[:, 128:158]


_BLK2 = 2048


def _tc_mlp(rows2, w1d, b1d, w2d, b2d, w3d, b3d, n_steps, step0):
    wspec = pl.BlockSpec((256, 256), lambda i: (0, 0))
    bspec = pl.BlockSpec((1, 256), lambda i: (0, 0))
    return pl.pallas_call(
        _mlp_body,
        grid=(n_steps,),
        in_specs=[
            pl.BlockSpec((_BLK2, 256), lambda i: (i + step0, 0)),
            wspec, bspec, wspec, bspec, wspec, bspec,
        ],
        out_specs=pl.BlockSpec((_BLK2, 2, 30), lambda i: (i, 0, 0)),
        out_shape=jax.ShapeDtypeStruct((n_steps * _BLK2, 2, 30), jnp.float32),
    )(rows2, w1d, b1d, w2d, b2d, w3d, b3d)


def _blockdiag(w, fan_in, fan_out):
    wp = jnp.zeros((_D, _D), jnp.float32).at[:fan_in, :fan_out].set(w)
    wd = jnp.zeros((256, 256), jnp.float32)
    return wd.at[:_D, :_D].set(wp).at[_D:, _D:].set(wp)


def kernel(textC, emoC, tableC, W1, b1, W2, b2, W3, b3):
    textC = textC.astype(jnp.int32)
    emoC = emoC.astype(jnp.int32)

    tidx3d = textC.reshape(_NW, _TCH_W, _CHUNK)
    eidx3d = emoC.T.reshape(_LE, _NW, _EB_W).transpose(1, 0, 2)

    rows = _sc_gather(tableC, tidx3d, eidx3d)

    w1d = _blockdiag(W1, _D, 100)
    w2d = _blockdiag(W2, 100, 60)
    w3d = _blockdiag(W3, 60, 30)

    def bpad(b, width):
        bp = jnp.zeros((1, _D), jnp.float32).at[0, :width].set(b)
        return jnp.concatenate([bp, bp], axis=1)

    b1d = bpad(b1, 100)
    b2d = bpad(b2, 60)
    b3d = bpad(b3, 30)

    rows2 = rows.reshape(_NROWS // 2, 256)
    n_text_steps = _NTEXT // (2 * _BLK2)
    out_text = _tc_mlp(rows2, w1d, b1d, w2d, b2d, w3d, b3d,
                       n_steps=n_text_steps, step0=0)
    out_emo = _tc_mlp(rows2, w1d, b1d, w2d, b2d, w3d, b3d,
                      n_steps=1, step0=n_text_steps)

    outputsC = out_text.reshape(_B, _S, 30)
    emo_out = out_emo.reshape(_B, 1, 30)
    return (outputsC, emo_out)


# speedup vs baseline: 1.2599x; 1.1262x over previous
import functools

import jax
import jax.numpy as jnp
from jax import lax
from jax.experimental import pallas as pl
from jax.experimental.pallas import tpu as pltpu
from jax.experimental.pallas import tpu_sc as plsc

_D = 128
_B = 4096
_S = 50
_LE = 20
_NW = 32
_CHUNK = 128

_NTEXT = _B * _S
_TCHUNKS = _NTEXT // _CHUNK
_TCH_W = _TCHUNKS // _NW
_EB_W = _B // _NW
_NROWS = _NTEXT + _B


def _sc_gather(table, tidx3d, eidx3d):
    mesh = plsc.VectorSubcoreMesh(core_axis_name="c", subcore_axis_name="s")

    @functools.partial(
        pl.kernel,
        mesh=mesh,
        out_type=jax.ShapeDtypeStruct((_NROWS, _D), jnp.float32),
        scratch_types=[
            pltpu.VMEM((_TCH_W, _CHUNK), jnp.int32),
            pltpu.VMEM((_LE, _CHUNK), jnp.int32),
            pltpu.VMEM((_CHUNK, _D), jnp.float32),
            pltpu.VMEM((_EB_W, _D), jnp.float32),
            pltpu.SemaphoreType.DMA,
        ],
    )
    def gather_kernel(table_hbm, tidx_hbm, eidx_hbm, out_hbm,
                      idx_v, eidx_v, rows_v, acc_v, sem):
        wid = lax.axis_index("s") * 2 + lax.axis_index("c")

        pltpu.sync_copy(tidx_hbm.at[wid], idx_v)

        def tbody(j, carry):
            pltpu.async_copy(table_hbm.at[idx_v.at[j]], rows_v, sem).wait()
            pltpu.sync_copy(
                rows_v, out_hbm.at[pl.ds((wid * _TCH_W + j) * _CHUNK, _CHUNK)])
            return carry

        lax.fori_loop(0, _TCH_W, tbody, 0)

        pltpu.sync_copy(eidx_hbm.at[wid], eidx_v)
        pltpu.async_copy(table_hbm.at[eidx_v.at[0]], acc_v, sem).wait()

        def ebody(l, carry):
            pltpu.async_copy(table_hbm.at[eidx_v.at[l]], acc_v, sem,
                             add=True).wait()
            return carry

        lax.fori_loop(1, _LE, ebody, 0)
        pltpu.sync_copy(acc_v, out_hbm.at[pl.ds(_NTEXT + wid * _EB_W, _EB_W)])

    return gather_kernel(table, tidx3d, eidx3d)


def _leaky(x):
    return jnp.maximum(x, 0.01 * x)


def _mlp_math(x, w1_ref, b1_ref, w2_ref, b2_ref, w3_ref, b3_ref):
    h = _leaky(jnp.dot(x, w1_ref[...], preferred_element_type=jnp.float32)
               + b1_ref[...])
    h = _leaky(jnp.dot(h, w2_ref[...], preferred_element_type=jnp.float32)
               + b2_ref[...])
    h = _leaky(jnp.dot(h, w3_ref[...], preferred_element_type=jnp.float32)
               + b3_ref[...])
    return h


def _mlp_text_body(x_ref, w1_ref, b1_ref, w2_ref, b2_ref, w3_ref, b3_ref,
                   o_ref):
    h = _mlp_math(x_ref[...], w1_ref, b1_ref, w2_ref, b2_ref, w3_ref, b3_ref)
    hb = h.reshape(_TB, _S // 2, 256)
    o = jnp.concatenate([hb[:, :, None, 0:30], hb[:, :, None, 128:158]],
                        axis=2)
    o_ref[...] = o.reshape(_TB, _S, 30)


def _mlp_emo_body(x_ref, w1_ref, b1_ref, w2_ref, b2_ref, w3_ref, b3_ref,
                  o_ref):
    h = _mlp_math(x_ref[...], w1_ref, b1_ref, w2_ref, b2_ref, w3_ref, b3_ref)
    o = jnp.concatenate([h[:, None, 0:30], h[:, None, 128:158]], axis=1)
    o_ref[...] = o.reshape(_B, 1, 30)


_TB = 128
_TBLK2 = _TB * _S // 2


def _tc_mlp_text(rows2, w1d, b1d, w2d, b2d, w3d, b3d):
    wspec = pl.BlockSpec((256, 256), lambda i: (0, 0))
    bspec = pl.BlockSpec((1, 256), lambda i: (0, 0))
    return pl.pallas_call(
        _mlp_text_body,
        grid=(_B // _TB,),
        in_specs=[
            pl.BlockSpec((_TBLK2, 256), lambda i: (i, 0)),
            wspec, bspec, wspec, bspec, wspec, bspec,
        ],
        out_specs=pl.BlockSpec((_TB, _S, 30), lambda i: (i, 0, 0)),
        out_shape=jax.ShapeDtypeStruct((_B, _S, 30), jnp.float32),
    )(rows2, w1d, b1d, w2d, b2d, w3d, b3d)


def _tc_mlp_emo(rows2, w1d, b1d, w2d, b2d, w3d, b3d):
    wspec = pl.BlockSpec((256, 256), lambda i: (0, 0))
    bspec = pl.BlockSpec((1, 256), lambda i: (0, 0))
    emo_start = _NTEXT // _B
    return pl.pallas_call(
        _mlp_emo_body,
        grid=(1,),
        in_specs=[
            pl.BlockSpec((_B // 2, 256), lambda i: (emo_start, 0)),
            wspec, bspec, wspec, bspec, wspec, bspec,
        ],
        out_specs=pl.BlockSpec((_B, 1, 30), lambda i: (0, 0, 0)),
        out_shape=jax.ShapeDtypeStruct((_B, 1, 30), jnp.float32),
    )(rows2, w1d, b1d, w2d, b2d, w3d, b3d)


def _blockdiag(w, fan_in, fan_out):
    wp = jnp.zeros((_D, _D), jnp.float32).at[:fan_in, :fan_out].set(w)
    wd = jnp.zeros((256, 256), jnp.float32)
    return wd.at[:_D, :_D].set(wp).at[_D:, _D:].set(wp)


def kernel(textC, emoC, tableC, W1, b1, W2, b2, W3, b3):
    textC = textC.astype(jnp.int32)
    emoC = emoC.astype(jnp.int32)

    tidx3d = textC.reshape(_NW, _TCH_W, _CHUNK)
    eidx3d = emoC.T.reshape(_LE, _NW, _EB_W).transpose(1, 0, 2)

    rows = _sc_gather(tableC, tidx3d, eidx3d)

    w1d = _blockdiag(W1, _D, 100)
    w2d = _blockdiag(W2, 100, 60)
    w3d = _blockdiag(W3, 60, 30)

    def bpad(b, width):
        bp = jnp.zeros((1, _D), jnp.float32).at[0, :width].set(b)
        return jnp.concatenate([bp, bp], axis=1)

    b1d = bpad(b1, 100)
    b2d = bpad(b2, 60)
    b3d = bpad(b3, 30)

    rows2 = rows.reshape(_NROWS // 2, 256)
    outputsC = _tc_mlp_text(rows2, w1d, b1d, w2d, b2d, w3d, b3d)
    emo_out = _tc_mlp_emo(rows2, w1d, b1d, w2d, b2d, w3d, b3d)
    return (outputsC, emo_out)


# speedup vs baseline: 1.2917x; 1.0252x over previous
import functools

import jax
import jax.numpy as jnp
from jax import lax
from jax.experimental import pallas as pl
from jax.experimental.pallas import tpu as pltpu
from jax.experimental.pallas import tpu_sc as plsc

_D = 128
_B = 4096
_S = 50
_LE = 20
_NW = 32
_CHUNK = 128

_NTEXT = _B * _S
_TCHUNKS = _NTEXT // _CHUNK
_TCH_W = _TCHUNKS // _NW
_EB_W = _B // _NW
_NROWS = _NTEXT + _B


_HCH = _CHUNK // 2


def _sc_gather(table, tidx3d, eidx3d):
    mesh = plsc.VectorSubcoreMesh(core_axis_name="c", subcore_axis_name="s")

    @functools.partial(
        pl.kernel,
        mesh=mesh,
        out_type=jax.ShapeDtypeStruct((_NROWS // 2, 2 * _D), jnp.float32),
        scratch_types=[
            pltpu.VMEM((2 * _TCH_W, _HCH), jnp.int32),
            pltpu.VMEM((2 * _LE, _HCH), jnp.int32),
            pltpu.VMEM((_HCH, 2 * _D), jnp.float32),
            pltpu.VMEM((_EB_W // 2, _D), jnp.float32),
            pltpu.VMEM((_EB_W // 2, _D), jnp.float32),
            pltpu.SemaphoreType.DMA,
        ],
    )
    def gather_kernel(table_hbm, tidx_hbm, eidx_hbm, out_hbm,
                      idx_v, eidx_v, rows_v, acc_l, acc_r, sem):
        wid = lax.axis_index("s") * 2 + lax.axis_index("c")

        pltpu.sync_copy(tidx_hbm.at[wid], idx_v)

        def tbody(j, carry):
            pltpu.async_copy(table_hbm.at[idx_v.at[2 * j]],
                             rows_v.at[:, 0:_D], sem).wait()
            pltpu.async_copy(table_hbm.at[idx_v.at[2 * j + 1]],
                             rows_v.at[:, _D:2 * _D], sem).wait()
            pltpu.sync_copy(
                rows_v, out_hbm.at[pl.ds((wid * _TCH_W + j) * _HCH, _HCH)])
            return carry

        lax.fori_loop(0, _TCH_W, tbody, 0)

        pltpu.sync_copy(eidx_hbm.at[wid], eidx_v)
        pltpu.async_copy(table_hbm.at[eidx_v.at[0]], acc_l, sem).wait()
        pltpu.async_copy(table_hbm.at[eidx_v.at[1]], acc_r, sem).wait()

        def ebody(l, carry):
            pltpu.async_copy(table_hbm.at[eidx_v.at[2 * l]],
                             acc_l, sem, add=True).wait()
            pltpu.async_copy(table_hbm.at[eidx_v.at[2 * l + 1]],
                             acc_r, sem, add=True).wait()
            return carry

        lax.fori_loop(1, _LE, ebody, 0)
        erow = _NTEXT // 2 + wid * (_EB_W // 2)
        pltpu.sync_copy(acc_l, out_hbm.at[pl.ds(erow, _EB_W // 2),
                                          pl.ds(0, _D)])
        pltpu.sync_copy(acc_r, out_hbm.at[pl.ds(erow, _EB_W // 2),
                                          pl.ds(_D, _D)])

    return gather_kernel(table, tidx3d, eidx3d)


def _leaky(x):
    return jnp.maximum(x, 0.01 * x)


def _mlp_math(x, w1_ref, b1_ref, w2_ref, b2_ref, w3_ref, b3_ref):
    h = _leaky(jnp.dot(x, w1_ref[...], preferred_element_type=jnp.float32)
               + b1_ref[...])
    h = _leaky(jnp.dot(h, w2_ref[...], preferred_element_type=jnp.float32)
               + b2_ref[...])
    h = _leaky(jnp.dot(h, w3_ref[...], preferred_element_type=jnp.float32)
               + b3_ref[...])
    return h


def _mlp_text_body(x_ref, w1_ref, b1_ref, w2_ref, b2_ref, w3_ref, b3_ref,
                   o_ref):
    h = _mlp_math(x_ref[...], w1_ref, b1_ref, w2_ref, b2_ref, w3_ref, b3_ref)
    hb = h.reshape(_TB, _S // 2, 256)
    o = jnp.concatenate([hb[:, :, None, 0:30], hb[:, :, None, 128:158]],
                        axis=2)
    o_ref[...] = o.reshape(_TB, _S, 30)


def _mlp_emo_body(x_ref, w1_ref, b1_ref, w2_ref, b2_ref, w3_ref, b3_ref,
                  o_ref):
    h = _mlp_math(x_ref[...], w1_ref, b1_ref, w2_ref, b2_ref, w3_ref, b3_ref)
    o = jnp.concatenate([h[:, None, 0:30], h[:, None, 128:158]], axis=1)
    o_ref[...] = o.reshape(_B, 1, 30)


_TB = 128
_TBLK2 = _TB * _S // 2


def _tc_mlp_text(rows2, w1d, b1d, w2d, b2d, w3d, b3d):
    wspec = pl.BlockSpec((256, 256), lambda i: (0, 0))
    bspec = pl.BlockSpec((1, 256), lambda i: (0, 0))
    return pl.pallas_call(
        _mlp_text_body,
        grid=(_B // _TB,),
        in_specs=[
            pl.BlockSpec((_TBLK2, 256), lambda i: (i, 0)),
            wspec, bspec, wspec, bspec, wspec, bspec,
        ],
        out_specs=pl.BlockSpec((_TB, _S, 30), lambda i: (i, 0, 0)),
        out_shape=jax.ShapeDtypeStruct((_B, _S, 30), jnp.float32),
    )(rows2, w1d, b1d, w2d, b2d, w3d, b3d)


def _tc_mlp_emo(rows2, w1d, b1d, w2d, b2d, w3d, b3d):
    wspec = pl.BlockSpec((256, 256), lambda i: (0, 0))
    bspec = pl.BlockSpec((1, 256), lambda i: (0, 0))
    emo_start = _NTEXT // _B
    return pl.pallas_call(
        _mlp_emo_body,
        grid=(1,),
        in_specs=[
            pl.BlockSpec((_B // 2, 256), lambda i: (emo_start, 0)),
            wspec, bspec, wspec, bspec, wspec, bspec,
        ],
        out_specs=pl.BlockSpec((_B, 1, 30), lambda i: (0, 0, 0)),
        out_shape=jax.ShapeDtypeStruct((_B, 1, 30), jnp.float32),
    )(rows2, w1d, b1d, w2d, b2d, w3d, b3d)


def _blockdiag(w, fan_in, fan_out):
    wp = jnp.zeros((_D, _D), jnp.float32).at[:fan_in, :fan_out].set(w)
    wd = jnp.zeros((256, 256), jnp.float32)
    return wd.at[:_D, :_D].set(wp).at[_D:, _D:].set(wp)


def kernel(textC, emoC, tableC, W1, b1, W2, b2, W3, b3):
    textC = textC.astype(jnp.int32)
    emoC = emoC.astype(jnp.int32)

    tidx = textC.reshape(_NW, _TCH_W, _HCH, 2).transpose(0, 1, 3, 2)
    tidx = tidx.reshape(_NW, 2 * _TCH_W, _HCH)
    eidx = emoC.reshape(_NW, _HCH, 2, _LE).transpose(0, 3, 2, 1)
    eidx = eidx.reshape(_NW, 2 * _LE, _HCH)

    rows2 = _sc_gather(tableC, tidx, eidx)

    w1d = _blockdiag(W1, _D, 100)
    w2d = _blockdiag(W2, 100, 60)
    w3d = _blockdiag(W3, 60, 30)

    def bpad(b, width):
        bp = jnp.zeros((1, _D), jnp.float32).at[0, :width].set(b)
        return jnp.concatenate([bp, bp], axis=1)

    b1d = bpad(b1, 100)
    b2d = bpad(b2, 60)
    b3d = bpad(b3, 30)

    outputsC = _tc_mlp_text(rows2, w1d, b1d, w2d, b2d, w3d, b3d)
    emo_out = _tc_mlp_emo(rows2, w1d, b1d, w2d, b2d, w3d, b3d)
    return (outputsC, emo_out)


# speedup vs baseline: 2.0902x; 1.6181x over previous
import functools

import jax
import jax.numpy as jnp
from jax import lax
from jax.experimental import pallas as pl
from jax.experimental.pallas import tpu as pltpu
from jax.experimental.pallas import tpu_sc as plsc

_D = 128
_B = 4096
_S = 50
_LE = 20
_NW = 32
_CHUNK = 128

_NTEXT = _B * _S
_EB_W = _B // _NW
_NROWS = _NTEXT + _B
_TCH = 25
_TROWS_W = _NTEXT // 2 // _NW


def _sc_gather(table, tidx3, eidx3):
    mesh = plsc.VectorSubcoreMesh(core_axis_name="c", subcore_axis_name="s")

    @functools.partial(
        pl.kernel,
        mesh=mesh,
        out_type=jax.ShapeDtypeStruct((_NROWS // 2, 2 * _D), jnp.float32),
        scratch_types=[
            pltpu.VMEM((2 * _TCH, _CHUNK), jnp.int32),
            pltpu.VMEM((2 * _LE, _EB_W // 2), jnp.int32),
            pltpu.VMEM((_CHUNK, 2 * _D), jnp.float32),
            pltpu.VMEM((_CHUNK, 2 * _D), jnp.float32),
            pltpu.VMEM((_EB_W // 2, _D), jnp.float32),
            pltpu.VMEM((_EB_W // 2, _D), jnp.float32),
            pltpu.SemaphoreType.DMA,
            pltpu.SemaphoreType.DMA,
            pltpu.SemaphoreType.DMA,
            pltpu.SemaphoreType.DMA,
            pltpu.SemaphoreType.DMA,
        ],
    )
    def gather_kernel(table_hbm, tidx_hbm, eidx_hbm, out_hbm,
                      idx_v, eidx_v, buf0, buf1, acc_l, acc_r,
                      g0, g1, s0, s1, es):
        wid = lax.axis_index("s") * 2 + lax.axis_index("c")
        tbase = wid * _TROWS_W

        pltpu.sync_copy(tidx_hbm.at[wid], idx_v)
        pltpu.sync_copy(eidx_hbm.at[wid], eidx_v)

        d_el = pltpu.async_copy(table_hbm.at[eidx_v.at[0]], acc_l, es)
        d_er = pltpu.async_copy(table_hbm.at[eidx_v.at[1]], acc_r, es)

        def fire_gathers(j, buf, sem):
            pltpu.async_copy(table_hbm.at[idx_v.at[2 * j]],
                             buf.at[:, 0:_D], sem)
            pltpu.async_copy(table_hbm.at[idx_v.at[2 * j + 1]],
                             buf.at[:, _D:2 * _D], sem)

        def wait_gathers(buf, sem):
            pltpu.make_async_copy(table_hbm.at[idx_v.at[0]],
                                  buf.at[:, 0:_D], sem).wait()
            pltpu.make_async_copy(table_hbm.at[idx_v.at[0]],
                                  buf.at[:, _D:2 * _D], sem).wait()

        fire_gathers(0, buf0, g0)
        fire_gathers(1, buf1, g1)

        d_el.wait()
        d_er.wait()
        eadds = []
        for l in range(1, _LE):
            eadds.append(pltpu.async_copy(table_hbm.at[eidx_v.at[2 * l]],
                                          acc_l, es, add=True))
            eadds.append(pltpu.async_copy(table_hbm.at[eidx_v.at[2 * l + 1]],
                                          acc_r, es, add=True))

        def slot(j, buf, gsem, ssem, fire_next):
            wait_gathers(buf, gsem)
            pltpu.async_copy(
                buf, out_hbm.at[pl.ds(tbase + j * _CHUNK, _CHUNK)],
                ssem).wait()
            if fire_next:
                @pl.when(j + 2 <= _TCH - 1)
                def _():
                    fire_gathers(j + 2, buf, gsem)

        def tloop(jj, carry):
            slot(2 * jj, buf0, g0, s0, fire_next=True)
            slot(2 * jj + 1, buf1, g1, s1, fire_next=True)
            return carry

        lax.fori_loop(0, (_TCH - 1) // 2, tloop, 0)
        slot(_TCH - 1, buf0, g0, s0, fire_next=False)

        for d in eadds:
            d.wait()
        erow = _NTEXT // 2 + wid * (_EB_W // 2)
        pltpu.sync_copy(acc_l, out_hbm.at[pl.ds(erow, _EB_W // 2),
                                          pl.ds(0, _D)])
        pltpu.sync_copy(acc_r, out_hbm.at[pl.ds(erow, _EB_W // 2),
                                          pl.ds(_D, _D)])

    return gather_kernel(table, tidx3, eidx3)


def _leaky(x):
    return jnp.maximum(x, 0.01 * x)


def _mlp_math(x, w1_ref, b1_ref, w2_ref, b2_ref, w3_ref, b3_ref):
    h = _leaky(jnp.dot(x, w1_ref[...], preferred_element_type=jnp.float32)
               + b1_ref[...])
    h = _leaky(jnp.dot(h, w2_ref[...], preferred_element_type=jnp.float32)
               + b2_ref[...])
    h = _leaky(jnp.dot(h, w3_ref[...], preferred_element_type=jnp.float32)
               + b3_ref[...])
    return h


def _mlp_text_body(x_ref, w1_ref, b1_ref, w2_ref, b2_ref, w3_ref, b3_ref,
                   o_ref):
    h = _mlp_math(x_ref[...], w1_ref, b1_ref, w2_ref, b2_ref, w3_ref, b3_ref)
    h3 = h.reshape(_TCH, _CHUNK, 256)
    o = jnp.concatenate([h3[:, None, :, 0:30], h3[:, None, :, 128:158]],
                        axis=1)
    o_ref[...] = o.reshape(_TB, _S, 30)


def _mlp_emo_body(x_ref, w1_ref, b1_ref, w2_ref, b2_ref, w3_ref, b3_ref,
                  o_ref):
    h = _mlp_math(x_ref[...], w1_ref, b1_ref, w2_ref, b2_ref, w3_ref, b3_ref)
    o = jnp.concatenate([h[:, None, 0:30], h[:, None, 128:158]], axis=1)
    o_ref[...] = o.reshape(_B, 1, 30)


_TB = 128
_TBLK2 = _TB * _S // 2


def _tc_mlp_text(rows2, w1d, b1d, w2d, b2d, w3d, b3d):
    wspec = pl.BlockSpec((256, 256), lambda i: (0, 0))
    bspec = pl.BlockSpec((1, 256), lambda i: (0, 0))
    return pl.pallas_call(
        _mlp_text_body,
        grid=(_B // _TB,),
        in_specs=[
            pl.BlockSpec((_TBLK2, 256), lambda i: (i, 0)),
            wspec, bspec, wspec, bspec, wspec, bspec,
        ],
        out_specs=pl.BlockSpec((_TB, _S, 30), lambda i: (i, 0, 0)),
        out_shape=jax.ShapeDtypeStruct((_B, _S, 30), jnp.float32),
    )(rows2, w1d, b1d, w2d, b2d, w3d, b3d)


def _tc_mlp_emo(rows2, w1d, b1d, w2d, b2d, w3d, b3d):
    wspec = pl.BlockSpec((256, 256), lambda i: (0, 0))
    bspec = pl.BlockSpec((1, 256), lambda i: (0, 0))
    emo_start = _NTEXT // _B
    return pl.pallas_call(
        _mlp_emo_body,
        grid=(1,),
        in_specs=[
            pl.BlockSpec((_B // 2, 256), lambda i: (emo_start, 0)),
            wspec, bspec, wspec, bspec, wspec, bspec,
        ],
        out_specs=pl.BlockSpec((_B, 1, 30), lambda i: (0, 0, 0)),
        out_shape=jax.ShapeDtypeStruct((_B, 1, 30), jnp.float32),
    )(rows2, w1d, b1d, w2d, b2d, w3d, b3d)


def _blockdiag(w, fan_in, fan_out):
    wp = jnp.zeros((_D, _D), jnp.float32).at[:fan_in, :fan_out].set(w)
    wd = jnp.zeros((256, 256), jnp.float32)
    return wd.at[:_D, :_D].set(wp).at[_D:, _D:].set(wp)


def kernel(textC, emoC, tableC, W1, b1, W2, b2, W3, b3):
    textC = textC.astype(jnp.int32)
    emoC = emoC.astype(jnp.int32)

    tidx3 = textC.reshape(_NW, 2 * _TCH, _CHUNK)
    eidx3 = emoC.reshape(_NW, _EB_W // 2, 2, _LE).transpose(0, 3, 2, 1)
    eidx3 = eidx3.reshape(_NW, 2 * _LE, _EB_W // 2)

    rows2 = _sc_gather(tableC, tidx3, eidx3)

    w1d = _blockdiag(W1, _D, 100)
    w2d = _blockdiag(W2, 100, 60)
    w3d = _blockdiag(W3, 60, 30)

    def bpad(b, width):
        bp = jnp.zeros((1, _D), jnp.float32).at[0, :width].set(b)
        return jnp.concatenate([bp, bp], axis=1)

    b1d = bpad(b1, 100)
    b2d = bpad(b2, 60)
    b3d = bpad(b3, 30)

    outputsC = _tc_mlp_text(rows2, w1d, b1d, w2d, b2d, w3d, b3d)
    emo_out = _tc_mlp_emo(rows2, w1d, b1d, w2d, b2d, w3d, b3d)
    return (outputsC, emo_out)


# speedup vs baseline: 2.1225x; 1.0154x over previous
import functools

import jax
import jax.numpy as jnp
from jax import lax
from jax.experimental import pallas as pl
from jax.experimental.pallas import tpu as pltpu
from jax.experimental.pallas import tpu_sc as plsc

_D = 128
_B = 4096
_S = 50
_LE = 20
_NW = 32
_CHUNK = 128

_NTEXT = _B * _S
_EB_W = _B // _NW
_NROWS = _NTEXT + _B
_TCH = 25
_TROWS_W = _NTEXT // 2 // _NW


def _sc_gather(table, tidx3, eidx3):
    mesh = plsc.VectorSubcoreMesh(core_axis_name="c", subcore_axis_name="s")

    @functools.partial(
        pl.kernel,
        mesh=mesh,
        out_type=jax.ShapeDtypeStruct((_NROWS // 2, 2 * _D), jnp.float32),
        scratch_types=[
            pltpu.VMEM((2 * _TCH, _CHUNK), jnp.int32),
            pltpu.VMEM((_LE, _EB_W), jnp.int32),
            pltpu.VMEM((_CHUNK, 2 * _D), jnp.float32),
            pltpu.VMEM((_CHUNK, 2 * _D), jnp.float32),
            pltpu.VMEM((_EB_W // 2, _D), jnp.float32),
            pltpu.VMEM((_EB_W // 2, _D), jnp.float32),
            pltpu.SemaphoreType.DMA,
            pltpu.SemaphoreType.DMA,
            pltpu.SemaphoreType.DMA,
            pltpu.SemaphoreType.DMA,
            pltpu.SemaphoreType.DMA,
        ],
    )
    def gather_kernel(table_hbm, tidx_hbm, eidx_hbm, out_hbm,
                      idx_v, eidx_v, buf0, buf1, acc_l, acc_r,
                      g0, g1, s0, s1, es):
        wid = lax.axis_index("s") * 2 + lax.axis_index("c")
        tbase = wid * _TROWS_W

        pltpu.sync_copy(tidx_hbm.at[wid], idx_v)
        pltpu.sync_copy(eidx_hbm.at[:, pl.ds(wid * _EB_W, _EB_W)], eidx_v)

        d_el = pltpu.async_copy(table_hbm.at[eidx_v.at[0, pl.ds(0, 64)]],
                                acc_l, es)
        d_er = pltpu.async_copy(table_hbm.at[eidx_v.at[0, pl.ds(64, 64)]],
                                acc_r, es)

        def fire_gathers(j, buf, sem):
            pltpu.async_copy(table_hbm.at[idx_v.at[2 * j]],
                             buf.at[:, 0:_D], sem)
            pltpu.async_copy(table_hbm.at[idx_v.at[2 * j + 1]],
                             buf.at[:, _D:2 * _D], sem)

        def wait_gathers(buf, sem):
            pltpu.make_async_copy(table_hbm.at[idx_v.at[0]],
                                  buf.at[:, 0:_D], sem).wait()
            pltpu.make_async_copy(table_hbm.at[idx_v.at[0]],
                                  buf.at[:, _D:2 * _D], sem).wait()

        fire_gathers(0, buf0, g0)
        fire_gathers(1, buf1, g1)

        d_el.wait()
        d_er.wait()
        eadds = []
        for l in range(1, _LE):
            eadds.append(pltpu.async_copy(
                table_hbm.at[eidx_v.at[l, pl.ds(0, 64)]],
                acc_l, es, add=True))
            eadds.append(pltpu.async_copy(
                table_hbm.at[eidx_v.at[l, pl.ds(64, 64)]],
                acc_r, es, add=True))

        def slot(j, buf, gsem, ssem, fire_next):
            wait_gathers(buf, gsem)
            pltpu.async_copy(
                buf, out_hbm.at[pl.ds(tbase + j * _CHUNK, _CHUNK)],
                ssem).wait()
            if fire_next:
                @pl.when(j + 2 <= _TCH - 1)
                def _():
                    fire_gathers(j + 2, buf, gsem)

        def tloop(jj, carry):
            slot(2 * jj, buf0, g0, s0, fire_next=True)
            slot(2 * jj + 1, buf1, g1, s1, fire_next=True)
            return carry

        lax.fori_loop(0, (_TCH - 1) // 2, tloop, 0)
        slot(_TCH - 1, buf0, g0, s0, fire_next=False)

        for d in eadds:
            d.wait()
        erow = _NTEXT // 2 + wid * (_EB_W // 2)
        pltpu.sync_copy(acc_l, out_hbm.at[pl.ds(erow, _EB_W // 2),
                                          pl.ds(0, _D)])
        pltpu.sync_copy(acc_r, out_hbm.at[pl.ds(erow, _EB_W // 2),
                                          pl.ds(_D, _D)])

    return gather_kernel(table, tidx3, eidx3)


def _leaky(x):
    return jnp.maximum(x, 0.01 * x)


def _mlp_math(x, w1_ref, b1_ref, w2_ref, b2_ref, w3_ref, b3_ref):
    h = _leaky(jnp.dot(x, w1_ref[...], preferred_element_type=jnp.float32)
               + b1_ref[...])
    h = _leaky(jnp.dot(h, w2_ref[...], preferred_element_type=jnp.float32)
               + b2_ref[...])
    h = _leaky(jnp.dot(h, w3_ref[...], preferred_element_type=jnp.float32)
               + b3_ref[...])
    return h


def _mlp_text_body(x_ref, w1_ref, b1_ref, w2_ref, b2_ref, w3_ref, b3_ref,
                   o_ref):
    h = _mlp_math(x_ref[...], w1_ref, b1_ref, w2_ref, b2_ref, w3_ref, b3_ref)
    h3 = h.reshape(_TCH, _CHUNK, 256)
    o = jnp.concatenate([h3[:, None, :, 0:30], h3[:, None, :, 128:158]],
                        axis=1)
    o_ref[...] = o.reshape(_TB, _S, 30)


def _mlp_emo_body(x_ref, w1_ref, b1_ref, w2_ref, b2_ref, w3_ref, b3_ref,
                  o_ref):
    h = _mlp_math(x_ref[...], w1_ref, b1_ref, w2_ref, b2_ref, w3_ref, b3_ref)
    h3 = h.reshape(_NW, _EB_W // 2, 256)
    o = jnp.concatenate([h3[:, None, :, 0:30], h3[:, None, :, 128:158]],
                        axis=1)
    o_ref[...] = o.reshape(_B, 1, 30)


_TB = 128
_TBLK2 = _TB * _S // 2


def _tc_mlp_text(rows2, w1d, b1d, w2d, b2d, w3d, b3d):
    wspec = pl.BlockSpec((256, 256), lambda i: (0, 0))
    bspec = pl.BlockSpec((1, 256), lambda i: (0, 0))
    return pl.pallas_call(
        _mlp_text_body,
        grid=(_B // _TB,),
        in_specs=[
            pl.BlockSpec((_TBLK2, 256), lambda i: (i, 0)),
            wspec, bspec, wspec, bspec, wspec, bspec,
        ],
        out_specs=pl.BlockSpec((_TB, _S, 30), lambda i: (i, 0, 0)),
        out_shape=jax.ShapeDtypeStruct((_B, _S, 30), jnp.float32),
    )(rows2, w1d, b1d, w2d, b2d, w3d, b3d)


def _tc_mlp_emo(rows2, w1d, b1d, w2d, b2d, w3d, b3d):
    wspec = pl.BlockSpec((256, 256), lambda i: (0, 0))
    bspec = pl.BlockSpec((1, 256), lambda i: (0, 0))
    emo_start = _NTEXT // _B
    return pl.pallas_call(
        _mlp_emo_body,
        grid=(1,),
        in_specs=[
            pl.BlockSpec((_B // 2, 256), lambda i: (emo_start, 0)),
            wspec, bspec, wspec, bspec, wspec, bspec,
        ],
        out_specs=pl.BlockSpec((_B, 1, 30), lambda i: (0, 0, 0)),
        out_shape=jax.ShapeDtypeStruct((_B, 1, 30), jnp.float32),
    )(rows2, w1d, b1d, w2d, b2d, w3d, b3d)


def _blockdiag(w, fan_in, fan_out):
    wp = jnp.zeros((_D, _D), jnp.float32).at[:fan_in, :fan_out].set(w)
    wd = jnp.zeros((256, 256), jnp.float32)
    return wd.at[:_D, :_D].set(wp).at[_D:, _D:].set(wp)


def kernel(textC, emoC, tableC, W1, b1, W2, b2, W3, b3):
    textC = textC.astype(jnp.int32)
    emoC = emoC.astype(jnp.int32)

    tidx3 = textC.reshape(_NW, 2 * _TCH, _CHUNK)
    eidxT = emoC.T

    rows2 = _sc_gather(tableC, tidx3, eidxT)

    w1d = _blockdiag(W1, _D, 100)
    w2d = _blockdiag(W2, 100, 60)
    w3d = _blockdiag(W3, 60, 30)

    def bpad(b, width):
        bp = jnp.zeros((1, _D), jnp.float32).at[0, :width].set(b)
        return jnp.concatenate([bp, bp], axis=1)

    b1d = bpad(b1, 100)
    b2d = bpad(b2, 60)
    b3d = bpad(b3, 30)

    outputsC = _tc_mlp_text(rows2, w1d, b1d, w2d, b2d, w3d, b3d)
    emo_out = _tc_mlp_emo(rows2, w1d, b1d, w2d, b2d, w3d, b3d)
    return (outputsC, emo_out)
